# Initial kernel scaffold; baseline (speedup 1.0000x reference)
import jax
import jax.numpy as jnp
from jax.experimental import pallas as pl


def kernel(samples, contexts, gamma, beta, priors):
    raise NotImplementedError("write your pallas kernel here")


# speedup vs baseline: 1.3932x; 1.3932x over previous
import functools

import jax
import jax.numpy as jnp
from jax import lax
from jax.experimental import pallas as pl
from jax.experimental.pallas import tpu as pltpu
from jax.experimental.pallas import tpu_sc as plsc

N = 16384
D = 128
NCTX = 26
C = 32
NC = 2
NS = 16
NW = NC * NS
RPW = N // NW
L = 16
EPS = 1e-3

_f32 = jnp.float32
_i32 = jnp.int32

def _rsqrt(x):
    i = plsc.bitcast(x, _i32)
    i = jnp.int32(0x5F3759DF) - lax.shift_right_logical(i, 1)
    y = plsc.bitcast(i, _f32)
    for _ in range(3):
        y = y * (1.5 - 0.5 * x * y * y)
    return y


def _stats(samples_hbm, ctx_hbm, osum, osq, ocnt, slab, ctxv, accs, accq, accc):
    wid = lax.axis_index("s") * NC + lax.axis_index("c")
    base = wid * RPW
    pltpu.sync_copy(samples_hbm.at[pl.ds(base, RPW), :], slab)
    pltpu.sync_copy(ctx_hbm.at[pl.ds(base, RPW)], ctxv)

    zeros = jnp.zeros((L,), _f32)

    def zero_row(c, carry):
        for j in range(D // L):
            accs[c, pl.ds(j * L, L)] = zeros
            accq[c, pl.ds(j * L, L)] = zeros
        accc[c, :] = zeros
        return carry

    lax.fori_loop(0, C, zero_row, 0)

    iota = lax.iota(_i32, L)
    ones = jnp.ones((L,), _f32)

    def row(r, carry):
        ctxb = plsc.load_gather(ctxv, [jnp.full((L,), r, _i32)])
        for j in range(D // L):
            x = slab[r, pl.ds(j * L, L)]
            col = iota + (j * L)
            plsc.addupdate_scatter(accs, [ctxb, col], x)
            plsc.addupdate_scatter(accq, [ctxb, col], x * x)
        plsc.addupdate_scatter(accc, [ctxb, iota], ones)
        return carry

    lax.fori_loop(0, RPW, row, 0)

    pltpu.sync_copy(accs, osum.at[:, wid, :])
    pltpu.sync_copy(accq, osq.at[:, wid, :])
    pltpu.sync_copy(accc, ocnt.at[:, wid, :])


def _apply(samples_hbm, ctx_hbm, gp_hbm, bp_hbm, psum, psq, pcnt, out_hbm,
           slab, ctxv, bs, bq, bc, sv, tv, gv, bv):
    wid = lax.axis_index("s") * NC + lax.axis_index("c")
    base = wid * RPW
    pltpu.sync_copy(samples_hbm.at[pl.ds(base, RPW), :], slab)
    pltpu.sync_copy(ctx_hbm.at[pl.ds(base, RPW)], ctxv)
    pltpu.sync_copy(gp_hbm, gv)
    pltpu.sync_copy(bp_hbm, bv)

    zeros = jnp.zeros((L,), _f32)

    def per_ctx(c, carry):
        pltpu.sync_copy(psum.at[c], bs)
        pltpu.sync_copy(psq.at[c], bq)
        pltpu.sync_copy(pcnt.at[c], bc)

        def red_cnt(t, acc):
            return acc + bc[t, :]

        cnt = lax.fori_loop(0, NW, red_cnt, zeros)
        cnt = jnp.maximum(cnt, 1.0)

        for j in range(D // L):
            sl = pl.ds(j * L, L)

            def red(t, acc):
                a, q = acc
                return (a + bs[t, sl], q + bq[t, sl])

            a, q = lax.fori_loop(0, NW, red, (zeros, zeros))
            mean = a / cnt
            var = q / cnt - mean * mean
            inv = _rsqrt(var + EPS)
            s = gv[c, sl] * inv
            sv[c, sl] = s
            tv[c, sl] = bv[c, sl] - mean * s
        return carry

    lax.fori_loop(0, C, per_ctx, 0)

    iota = lax.iota(_i32, L)

    def row(r, carry):
        ctxb = plsc.load_gather(ctxv, [jnp.full((L,), r, _i32)])
        for j in range(D // L):
            sl = pl.ds(j * L, L)
            col = iota + (j * L)
            x = slab[r, sl]
            s = plsc.load_gather(sv, [ctxb, col])
            t = plsc.load_gather(tv, [ctxb, col])
            slab[r, sl] = x * s + t
        return carry

    lax.fori_loop(0, RPW, row, 0)

    pltpu.sync_copy(slab, out_hbm.at[pl.ds(base, RPW), :])


@functools.cache
def _build():
    mesh = plsc.VectorSubcoreMesh(
        core_axis_name="c", subcore_axis_name="s", num_cores=NC, num_subcores=NS
    )
    params = pltpu.CompilerParams(needs_layout_passes=False)
    stats = pl.kernel(
        _stats,
        compiler_params=params,
        out_type=(
            jax.ShapeDtypeStruct((C, NW, D), _f32),
            jax.ShapeDtypeStruct((C, NW, D), _f32),
            jax.ShapeDtypeStruct((C, NW, L), _f32),
        ),
        mesh=mesh,
        scratch_types=[
            pltpu.VMEM((RPW, D), _f32),
            pltpu.VMEM((RPW,), _i32),
            pltpu.VMEM((C, D), _f32),
            pltpu.VMEM((C, D), _f32),
            pltpu.VMEM((C, L), _f32),
        ],
    )
    apply_ = pl.kernel(
        _apply,
        compiler_params=params,
        out_type=jax.ShapeDtypeStruct((N, D), _f32),
        mesh=mesh,
        scratch_types=[
            pltpu.VMEM((RPW, D), _f32),
            pltpu.VMEM((RPW,), _i32),
            pltpu.VMEM((NW, D), _f32),
            pltpu.VMEM((NW, D), _f32),
            pltpu.VMEM((NW, L), _f32),
            pltpu.VMEM((C, D), _f32),
            pltpu.VMEM((C, D), _f32),
            pltpu.VMEM((C, D), _f32),
            pltpu.VMEM((C, D), _f32),
        ],
    )
    return stats, apply_


@jax.jit
def kernel(samples, contexts, gamma, beta, priors):
    _stats_k, _apply_k = _build()
    ctx = contexts.astype(_i32)
    invp = 1.0 / jnp.sqrt(priors)
    gp = jnp.pad(gamma * invp[:, None], ((0, C - NCTX), (0, 0)))
    bp = jnp.pad(beta * invp[:, None], ((0, C - NCTX), (0, 0)))
    s, q, n = _stats_k(samples, ctx)
    return _apply_k(samples, ctx, gp, bp, s, q, n)


# speedup vs baseline: 2.9489x; 2.1166x over previous
import functools

import jax
import jax.numpy as jnp
from jax import lax
from jax.experimental import pallas as pl
from jax.experimental.pallas import tpu as pltpu
from jax.experimental.pallas import tpu_sc as plsc

N = 16384
D = 128
NCTX = 26
C = 32
NC = 2
NS = 16
NW = NC * NS
RPW = N // NW
L = 16
EPS = 1e-3

_f32 = jnp.float32
_i32 = jnp.int32


def _rsqrt(x):
    i = plsc.bitcast(x, _i32)
    i = jnp.int32(0x5F3759DF) - lax.shift_right_logical(i, 1)
    y = plsc.bitcast(i, _f32)
    for _ in range(3):
        y = y * (1.5 - 0.5 * x * y * y)
    return y


def _wid():
    return lax.axis_index("s") * NC + lax.axis_index("c")


def _stats(samples_hbm, ctx_hbm, osum, osq, ocnt, slab, ctxv, accs, accq, accc):
    base = _wid() * RPW
    pltpu.sync_copy(samples_hbm.at[pl.ds(base, RPW), :], slab)
    pltpu.sync_copy(ctx_hbm.at[pl.ds(base, RPW)], ctxv)

    zeros = jnp.zeros((L,), _f32)

    def zero_row(c, carry):
        for j in range(D // L):
            accs[c, pl.ds(j * L, L)] = zeros
            accq[c, pl.ds(j * L, L)] = zeros
        accc[c, :] = zeros
        return carry

    lax.fori_loop(0, C, zero_row, 0, unroll=4)

    iota = lax.iota(_i32, L)
    ones = jnp.ones((L,), _f32)

    def row(r, carry):
        ctxb = plsc.load_gather(ctxv, [jnp.full((L,), r, _i32)])
        for j in range(D // L):
            x = slab[r, pl.ds(j * L, L)]
            col = iota + (j * L)
            plsc.addupdate_scatter(accs, [ctxb, col], x)
            plsc.addupdate_scatter(accq, [ctxb, col], x * x)
        plsc.addupdate_scatter(accc, [ctxb, iota], ones)
        return carry

    lax.fori_loop(0, RPW, row, 0, unroll=4)

    wid = _wid()
    pltpu.sync_copy(accs, osum.at[:, wid, :])
    pltpu.sync_copy(accq, osq.at[:, wid, :])
    pltpu.sync_copy(accc, ocnt.at[:, wid, :])


def _combine(psum, psq, pcnt, gp_hbm, bp_hbm, oscale, oshift,
             bs, bq, bc, gv, bv, sv, tv):
    c = _wid()
    pltpu.sync_copy(psum.at[c], bs)
    pltpu.sync_copy(psq.at[c], bq)
    pltpu.sync_copy(pcnt.at[c], bc)
    pltpu.sync_copy(gp_hbm.at[c], gv)
    pltpu.sync_copy(bp_hbm.at[c], bv)

    zeros = jnp.zeros((L,), _f32)

    def red_cnt(t, acc):
        return acc + bc[t, :]

    cnt = lax.fori_loop(0, NW, red_cnt, zeros, unroll=8)
    cnt = jnp.maximum(cnt, 1.0)

    for j in range(D // L):
        sl = pl.ds(j * L, L)

        def red(t, acc):
            a, q = acc
            return (a + bs[t, sl], q + bq[t, sl])

        a, q = lax.fori_loop(0, NW, red, (zeros, zeros), unroll=8)
        mean = a / cnt
        var = q / cnt - mean * mean
        inv = _rsqrt(var + EPS)
        s = gv[sl] * inv
        sv[sl] = s
        tv[sl] = bv[sl] - mean * s

    pltpu.sync_copy(sv, oscale.at[c])
    pltpu.sync_copy(tv, oshift.at[c])


def _apply(samples_hbm, ctx_hbm, scale_hbm, shift_hbm, out_hbm,
           slab, ctxv, sv, tv):
    base = _wid() * RPW
    pltpu.sync_copy(samples_hbm.at[pl.ds(base, RPW), :], slab)
    pltpu.sync_copy(ctx_hbm.at[pl.ds(base, RPW)], ctxv)
    pltpu.sync_copy(scale_hbm, sv)
    pltpu.sync_copy(shift_hbm, tv)

    iota = lax.iota(_i32, L)

    def row(r, carry):
        ctxb = plsc.load_gather(ctxv, [jnp.full((L,), r, _i32)])
        for j in range(D // L):
            sl = pl.ds(j * L, L)
            col = iota + (j * L)
            x = slab[r, sl]
            s = plsc.load_gather(sv, [ctxb, col])
            t = plsc.load_gather(tv, [ctxb, col])
            slab[r, sl] = x * s + t
        return carry

    lax.fori_loop(0, RPW, row, 0, unroll=4)

    pltpu.sync_copy(slab, out_hbm.at[pl.ds(base, RPW), :])


@functools.cache
def _build():
    mesh = plsc.VectorSubcoreMesh(
        core_axis_name="c", subcore_axis_name="s", num_cores=NC, num_subcores=NS
    )
    params = pltpu.CompilerParams(needs_layout_passes=False)
    stats = pl.kernel(
        _stats,
        compiler_params=params,
        out_type=(
            jax.ShapeDtypeStruct((C, NW, D), _f32),
            jax.ShapeDtypeStruct((C, NW, D), _f32),
            jax.ShapeDtypeStruct((C, NW, L), _f32),
        ),
        mesh=mesh,
        scratch_types=[
            pltpu.VMEM((RPW, D), _f32),
            pltpu.VMEM((RPW,), _i32),
            pltpu.VMEM((C, D), _f32),
            pltpu.VMEM((C, D), _f32),
            pltpu.VMEM((C, L), _f32),
        ],
    )
    combine = pl.kernel(
        _combine,
        compiler_params=params,
        out_type=(
            jax.ShapeDtypeStruct((C, D), _f32),
            jax.ShapeDtypeStruct((C, D), _f32),
        ),
        mesh=mesh,
        scratch_types=[
            pltpu.VMEM((NW, D), _f32),
            pltpu.VMEM((NW, D), _f32),
            pltpu.VMEM((NW, L), _f32),
            pltpu.VMEM((D,), _f32),
            pltpu.VMEM((D,), _f32),
            pltpu.VMEM((D,), _f32),
            pltpu.VMEM((D,), _f32),
        ],
    )
    apply_ = pl.kernel(
        _apply,
        compiler_params=params,
        out_type=jax.ShapeDtypeStruct((N, D), _f32),
        mesh=mesh,
        scratch_types=[
            pltpu.VMEM((RPW, D), _f32),
            pltpu.VMEM((RPW,), _i32),
            pltpu.VMEM((C, D), _f32),
            pltpu.VMEM((C, D), _f32),
        ],
    )
    return stats, combine, apply_


@jax.jit
def kernel(samples, contexts, gamma, beta, priors):
    _stats_k, _combine_k, _apply_k = _build()
    ctx = contexts.astype(_i32)
    invp = 1.0 / jnp.sqrt(priors)
    gp = jnp.pad(gamma * invp[:, None], ((0, C - NCTX), (0, 0)))
    bp = jnp.pad(beta * invp[:, None], ((0, C - NCTX), (0, 0)))
    s, q, n = _stats_k(samples, ctx)
    scale, shift = _combine_k(s, q, n, gp, bp)
    return _apply_k(samples, ctx, scale, shift)


# speedup vs baseline: 4.4968x; 1.5249x over previous
import functools

import jax
import jax.numpy as jnp
from jax import lax
from jax.experimental import pallas as pl
from jax.experimental.pallas import tpu as pltpu
from jax.experimental.pallas import tpu_sc as plsc

N = 16384
D = 128
NCTX = 26
C = 32
NC = 2
NS = 16
NW = NC * NS
RPW = N // NW
L = 16
EPS = 1e-3

_f32 = jnp.float32
_i32 = jnp.int32


def _rsqrt(x):
    i = plsc.bitcast(x, _i32)
    i = jnp.int32(0x5F3759DF) - lax.shift_right_logical(i, 1)
    y = plsc.bitcast(i, _f32)
    for _ in range(3):
        y = y * (1.5 - 0.5 * x * y * y)
    return y


def _wid():
    return lax.axis_index("s") * NC + lax.axis_index("c")


def _stats(samples_hbm, ctx_hbm, osum, osq, ocnt, slab, ctxv, accs, accq, accc):
    base = _wid() * RPW
    pltpu.sync_copy(samples_hbm.at[pl.ds(base, RPW), :], slab)
    pltpu.sync_copy(ctx_hbm.at[pl.ds(base, RPW)], ctxv)

    zeros = jnp.zeros((L,), _f32)

    @plsc.parallel_loop(0, C, unroll=4)
    def _zero(c):
        for j in range(D // L):
            accs[c, pl.ds(j * L, L)] = zeros
            accq[c, pl.ds(j * L, L)] = zeros
        accc[c, :] = zeros

    iota = lax.iota(_i32, L)
    ones = jnp.ones((L,), _f32)

    @plsc.parallel_loop(0, RPW, unroll=4)
    def _row(r):
        ctxb = plsc.load_gather(ctxv, [jnp.full((L,), r, _i32)])
        for j in range(D // L):
            x = slab[r, pl.ds(j * L, L)]
            col = iota + (j * L)
            plsc.addupdate_scatter(accs, [ctxb, col], x)
            plsc.addupdate_scatter(accq, [ctxb, col], x * x)
        plsc.addupdate_scatter(accc, [ctxb, iota], ones)

    wid = _wid()
    pltpu.sync_copy(accs, osum.at[:, wid, :])
    pltpu.sync_copy(accq, osq.at[:, wid, :])
    pltpu.sync_copy(accc, ocnt.at[:, wid, :])


def _combine(psum, psq, pcnt, gp_hbm, bp_hbm, oscale, oshift,
             bs, bq, bc, gv, bv, sv, tv):
    c = _wid()
    pltpu.sync_copy(psum.at[c], bs)
    pltpu.sync_copy(psq.at[c], bq)
    pltpu.sync_copy(pcnt.at[c], bc)
    pltpu.sync_copy(gp_hbm.at[c], gv)
    pltpu.sync_copy(bp_hbm.at[c], bv)

    zeros = jnp.zeros((L,), _f32)

    def red_cnt(t, acc):
        return acc + bc[t, :]

    cnt = lax.fori_loop(0, NW, red_cnt, zeros, unroll=8)
    cnt = jnp.maximum(cnt, 1.0)

    for j in range(D // L):
        sl = pl.ds(j * L, L)

        def red(t, acc):
            a, q = acc
            return (a + bs[t, sl], q + bq[t, sl])

        a, q = lax.fori_loop(0, NW, red, (zeros, zeros), unroll=8)
        mean = a / cnt
        var = q / cnt - mean * mean
        inv = _rsqrt(var + EPS)
        s = gv[sl] * inv
        sv[sl] = s
        tv[sl] = bv[sl] - mean * s

    pltpu.sync_copy(sv, oscale.at[c])
    pltpu.sync_copy(tv, oshift.at[c])


def _apply(samples_hbm, ctx_hbm, scale_hbm, shift_hbm, out_hbm,
           slab, ctxv, sv, tv):
    base = _wid() * RPW
    pltpu.sync_copy(samples_hbm.at[pl.ds(base, RPW), :], slab)
    pltpu.sync_copy(ctx_hbm.at[pl.ds(base, RPW)], ctxv)
    pltpu.sync_copy(scale_hbm, sv)
    pltpu.sync_copy(shift_hbm, tv)

    iota = lax.iota(_i32, L)

    @plsc.parallel_loop(0, RPW, unroll=4)
    def _row(r):
        ctxb = plsc.load_gather(ctxv, [jnp.full((L,), r, _i32)])
        for j in range(D // L):
            sl = pl.ds(j * L, L)
            col = iota + (j * L)
            x = slab[r, sl]
            s = plsc.load_gather(sv, [ctxb, col])
            t = plsc.load_gather(tv, [ctxb, col])
            slab[r, sl] = x * s + t

    pltpu.sync_copy(slab, out_hbm.at[pl.ds(base, RPW), :])


@functools.cache
def _build():
    mesh = plsc.VectorSubcoreMesh(
        core_axis_name="c", subcore_axis_name="s", num_cores=NC, num_subcores=NS
    )
    params = pltpu.CompilerParams(needs_layout_passes=False)
    stats = pl.kernel(
        _stats,
        compiler_params=params,
        out_type=(
            jax.ShapeDtypeStruct((C, NW, D), _f32),
            jax.ShapeDtypeStruct((C, NW, D), _f32),
            jax.ShapeDtypeStruct((C, NW, L), _f32),
        ),
        mesh=mesh,
        scratch_types=[
            pltpu.VMEM((RPW, D), _f32),
            pltpu.VMEM((RPW,), _i32),
            pltpu.VMEM((C, D), _f32),
            pltpu.VMEM((C, D), _f32),
            pltpu.VMEM((C, L), _f32),
        ],
    )
    combine = pl.kernel(
        _combine,
        compiler_params=params,
        out_type=(
            jax.ShapeDtypeStruct((C, D), _f32),
            jax.ShapeDtypeStruct((C, D), _f32),
        ),
        mesh=mesh,
        scratch_types=[
            pltpu.VMEM((NW, D), _f32),
            pltpu.VMEM((NW, D), _f32),
            pltpu.VMEM((NW, L), _f32),
            pltpu.VMEM((D,), _f32),
            pltpu.VMEM((D,), _f32),
            pltpu.VMEM((D,), _f32),
            pltpu.VMEM((D,), _f32),
        ],
    )
    apply_ = pl.kernel(
        _apply,
        compiler_params=params,
        out_type=jax.ShapeDtypeStruct((N, D), _f32),
        mesh=mesh,
        scratch_types=[
            pltpu.VMEM((RPW, D), _f32),
            pltpu.VMEM((RPW,), _i32),
            pltpu.VMEM((C, D), _f32),
            pltpu.VMEM((C, D), _f32),
        ],
    )
    return stats, combine, apply_


@jax.jit
def kernel(samples, contexts, gamma, beta, priors):
    _stats_k, _combine_k, _apply_k = _build()
    ctx = contexts.astype(_i32)
    invp = 1.0 / jnp.sqrt(priors)
    gp = jnp.pad(gamma * invp[:, None], ((0, C - NCTX), (0, 0)))
    bp = jnp.pad(beta * invp[:, None], ((0, C - NCTX), (0, 0)))
    s, q, n = _stats_k(samples, ctx)
    scale, shift = _combine_k(s, q, n, gp, bp)
    return _apply_k(samples, ctx, scale, shift)


# speedup vs baseline: 4.5088x; 1.0027x over previous
import functools

import jax
import jax.numpy as jnp
from jax import lax
from jax.experimental import pallas as pl
from jax.experimental.pallas import tpu as pltpu
from jax.experimental.pallas import tpu_sc as plsc

N = 16384
D = 128
NCTX = 26
C = 32
NC = 2
NS = 16
NW = NC * NS
RPW = N // NW
L = 16
EPS = 1e-3

_f32 = jnp.float32
_i32 = jnp.int32


def _rsqrt(x):
    i = plsc.bitcast(x, _i32)
    i = jnp.int32(0x5F3759DF) - lax.shift_right_logical(i, 1)
    y = plsc.bitcast(i, _f32)
    for _ in range(3):
        y = y * (1.5 - 0.5 * x * y * y)
    return y


def _wid():
    return lax.axis_index("s") * NC + lax.axis_index("c")


def _stats(samples_hbm, ctx_hbm, osum, osq, ocnt, slab, ctxv, accs, accq, accc):
    base = _wid() * RPW
    pltpu.sync_copy(samples_hbm.at[pl.ds(base, RPW), :], slab)
    pltpu.sync_copy(ctx_hbm.at[pl.ds(base, RPW)], ctxv)

    zeros = jnp.zeros((L,), _f32)

    @plsc.parallel_loop(0, C, unroll=4)
    def _zero(c):
        for j in range(D // L):
            accs[c, pl.ds(j * L, L)] = zeros
            accq[c, pl.ds(j * L, L)] = zeros
        accc[c, :] = zeros

    iota = lax.iota(_i32, L)
    ones = jnp.ones((L,), _f32)

    @plsc.parallel_loop(0, RPW, unroll=4)
    def _row(r):
        ctxb = plsc.load_gather(ctxv, [jnp.full((L,), r, _i32)])
        for j in range(D // L):
            x = slab[r, pl.ds(j * L, L)]
            col = iota + (j * L)
            plsc.addupdate_scatter(accs, [ctxb, col], x)
            plsc.addupdate_scatter(accq, [ctxb, col], x * x)
        plsc.addupdate_scatter(accc, [ctxb, iota], ones)

    wid = _wid()
    pltpu.sync_copy(accs, osum.at[:, wid, :])
    pltpu.sync_copy(accq, osq.at[:, wid, :])
    pltpu.sync_copy(accc, ocnt.at[:, wid, :])


def _apply(samples_hbm, ctx_hbm, gp_hbm, bp_hbm, psum, psq, pcnt, out_hbm,
           slab, ctxv, sv, tv, bs, bq, bc, grow, brow, srow, trow,
           shs, sht, sem_slab, sem_ctx):
    core = lax.axis_index("c")
    sid = lax.axis_index("s")
    base = (sid * NC + core) * RPW
    cp_slab = pltpu.async_copy(samples_hbm.at[pl.ds(base, RPW), :], slab, sem_slab)
    cp_ctx = pltpu.async_copy(ctx_hbm.at[pl.ds(base, RPW)], ctxv, sem_ctx)

    zeros = jnp.zeros((L,), _f32)

    for k in range(2):
        c = sid * 2 + k
        pltpu.sync_copy(psum.at[c], bs)
        pltpu.sync_copy(psq.at[c], bq)
        pltpu.sync_copy(pcnt.at[c], bc)
        pltpu.sync_copy(gp_hbm.at[c], grow)
        pltpu.sync_copy(bp_hbm.at[c], brow)

        def red_cnt(t, acc):
            return acc + bc[t, :]

        cnt = lax.fori_loop(0, NW, red_cnt, zeros, unroll=8)
        cnt = jnp.maximum(cnt, 1.0)

        for j in range(D // L):
            sl = pl.ds(j * L, L)

            def red(t, acc):
                a, q = acc
                return (a + bs[t, sl], q + bq[t, sl])

            a, q = lax.fori_loop(0, NW, red, (zeros, zeros), unroll=8)
            mean = a / cnt
            var = q / cnt - mean * mean
            inv = _rsqrt(var + EPS)
            s = grow[sl] * inv
            srow[sl] = s
            trow[sl] = brow[sl] - mean * s

        pltpu.sync_copy(srow, shs.at[c])
        pltpu.sync_copy(trow, sht.at[c])

    plsc.subcore_barrier()
    pltpu.sync_copy(shs, sv)
    pltpu.sync_copy(sht, tv)

    cp_slab.wait()
    cp_ctx.wait()

    iota = lax.iota(_i32, L)

    @plsc.parallel_loop(0, RPW, unroll=4)
    def _row(r):
        ctxb = plsc.load_gather(ctxv, [jnp.full((L,), r, _i32)])
        for j in range(D // L):
            sl = pl.ds(j * L, L)
            col = iota + (j * L)
            x = slab[r, sl]
            s = plsc.load_gather(sv, [ctxb, col])
            t = plsc.load_gather(tv, [ctxb, col])
            slab[r, sl] = x * s + t

    pltpu.sync_copy(slab, out_hbm.at[pl.ds(base, RPW), :])


@functools.cache
def _build():
    mesh = plsc.VectorSubcoreMesh(
        core_axis_name="c", subcore_axis_name="s", num_cores=NC, num_subcores=NS
    )
    params = pltpu.CompilerParams(needs_layout_passes=False)
    stats = pl.kernel(
        _stats,
        compiler_params=params,
        out_type=(
            jax.ShapeDtypeStruct((C, NW, D), _f32),
            jax.ShapeDtypeStruct((C, NW, D), _f32),
            jax.ShapeDtypeStruct((C, NW, L), _f32),
        ),
        mesh=mesh,
        scratch_types=[
            pltpu.VMEM((RPW, D), _f32),
            pltpu.VMEM((RPW,), _i32),
            pltpu.VMEM((C, D), _f32),
            pltpu.VMEM((C, D), _f32),
            pltpu.VMEM((C, L), _f32),
        ],
    )
    apply_ = pl.kernel(
        _apply,
        compiler_params=params,
        out_type=jax.ShapeDtypeStruct((N, D), _f32),
        mesh=mesh,
        scratch_types=[
            pltpu.VMEM((RPW, D), _f32),
            pltpu.VMEM((RPW,), _i32),
            pltpu.VMEM((C, D), _f32),
            pltpu.VMEM((C, D), _f32),
            pltpu.VMEM((NW, D), _f32),
            pltpu.VMEM((NW, D), _f32),
            pltpu.VMEM((NW, L), _f32),
            pltpu.VMEM((D,), _f32),
            pltpu.VMEM((D,), _f32),
            pltpu.VMEM((D,), _f32),
            pltpu.VMEM((D,), _f32),
            pltpu.VMEM_SHARED((C, D), _f32),
            pltpu.VMEM_SHARED((C, D), _f32),
            pltpu.SemaphoreType.DMA,
            pltpu.SemaphoreType.DMA,
        ],
    )
    return stats, apply_


@jax.jit
def kernel(samples, contexts, gamma, beta, priors):
    _stats_k, _apply_k = _build()
    ctx = contexts.astype(_i32)
    invp = 1.0 / jnp.sqrt(priors)
    gp = jnp.pad(gamma * invp[:, None], ((0, C - NCTX), (0, 0)))
    bp = jnp.pad(beta * invp[:, None], ((0, C - NCTX), (0, 0)))
    s, q, n = _stats_k(samples, ctx)
    return _apply_k(samples, ctx, gp, bp, s, q, n)


# speedup vs baseline: 4.7774x; 1.0596x over previous
import functools

import jax
import jax.numpy as jnp
from jax import lax
from jax.experimental import pallas as pl
from jax.experimental.pallas import tpu as pltpu
from jax.experimental.pallas import tpu_sc as plsc

N = 16384
D = 128
NCTX = 26
C = 32
NC = 2
NS = 16
NW = NC * NS
RPW = N // NW
L = 16
EPS = 1e-3

_f32 = jnp.float32
_i32 = jnp.int32


def _rsqrt(x):
    i = plsc.bitcast(x, _i32)
    i = jnp.int32(0x5F3759DF) - lax.shift_right_logical(i, 1)
    y = plsc.bitcast(i, _f32)
    for _ in range(3):
        y = y * (1.5 - 0.5 * x * y * y)
    return y


def _wid():
    return lax.axis_index("s") * NC + lax.axis_index("c")


def _stats(samples_hbm, ctx_hbm, osum, osq, ocnt, slab, ctxv, accs, accq, accc):
    base = _wid() * RPW
    pltpu.sync_copy(samples_hbm.at[pl.ds(base, RPW), :], slab)
    pltpu.sync_copy(ctx_hbm.at[pl.ds(base, RPW)], ctxv)

    zeros = jnp.zeros((L,), _f32)

    @plsc.parallel_loop(0, C, unroll=4)
    def _zero(c):
        for j in range(D // L):
            accs[c, pl.ds(j * L, L)] = zeros
            accq[c, pl.ds(j * L, L)] = zeros
        accc[c, :] = zeros

    iota = lax.iota(_i32, L)
    ones = jnp.ones((L,), _f32)

    @plsc.parallel_loop(0, RPW, unroll=4)
    def _row(r):
        ctxb = plsc.load_gather(ctxv, [jnp.full((L,), r, _i32)])
        for j in range(D // L):
            x = slab[r, pl.ds(j * L, L)]
            col = iota + (j * L)
            plsc.addupdate_scatter(accs, [ctxb, col], x)
            plsc.addupdate_scatter(accq, [ctxb, col], x * x)
        plsc.addupdate_scatter(accc, [ctxb, iota], ones)

    wid = _wid()
    pltpu.sync_copy(accs, osum.at[:, wid, :])
    pltpu.sync_copy(accq, osq.at[:, wid, :])
    pltpu.sync_copy(accc, ocnt.at[:, wid, :])


def _apply(samples_hbm, ctx_hbm, gamma_hbm, beta_hbm, priors_hbm,
           psum, psq, pcnt, out_hbm,
           slab, ctxv, sv, tv, bs, bq, bc, grow, brow, pv, srow, trow,
           shs, sht, sem_slab, sem_ctx, sem_cmb):
    core = lax.axis_index("c")
    sid = lax.axis_index("s")
    base = (sid * NC + core) * RPW
    cp_slab = pltpu.async_copy(samples_hbm.at[pl.ds(base, RPW), :], slab, sem_slab)
    cp_ctx = pltpu.async_copy(ctx_hbm.at[pl.ds(base, RPW)], ctxv, sem_ctx)

    cps = []
    for k in range(2):
        c = sid * 2 + k
        cg = jnp.minimum(c, NCTX - 1)
        cps.append(pltpu.async_copy(psum.at[c], bs.at[k], sem_cmb))
        cps.append(pltpu.async_copy(psq.at[c], bq.at[k], sem_cmb))
        cps.append(pltpu.async_copy(pcnt.at[c], bc.at[k], sem_cmb))
        cps.append(pltpu.async_copy(gamma_hbm.at[cg], grow.at[k], sem_cmb))
        cps.append(pltpu.async_copy(beta_hbm.at[cg], brow.at[k], sem_cmb))
    cps.append(pltpu.async_copy(priors_hbm, pv, sem_cmb))
    for cp in cps:
        cp.wait()

    zeros = jnp.zeros((L,), _f32)

    for k in range(2):
        c = sid * 2 + k
        cg = jnp.minimum(c, NCTX - 1)
        invp = _rsqrt(plsc.load_gather(pv, [jnp.full((L,), cg, _i32)]))

        def red_cnt(t, acc):
            return acc + bc[k, t, :]

        cnt = lax.fori_loop(0, NW, red_cnt, zeros, unroll=8)
        cnt = jnp.maximum(cnt, 1.0)

        for j in range(D // L):
            sl = pl.ds(j * L, L)

            def red(t, acc):
                a, q = acc
                return (a + bs[k, t, sl], q + bq[k, t, sl])

            a, q = lax.fori_loop(0, NW, red, (zeros, zeros), unroll=8)
            mean = a / cnt
            var = q / cnt - mean * mean
            inv = _rsqrt(var + EPS)
            s = grow[k, sl] * inv * invp
            srow[sl] = s
            trow[sl] = brow[k, sl] * invp - mean * s

        pltpu.sync_copy(srow, shs.at[c])
        pltpu.sync_copy(trow, sht.at[c])

    plsc.subcore_barrier()
    pltpu.sync_copy(shs, sv)
    pltpu.sync_copy(sht, tv)

    cp_slab.wait()
    cp_ctx.wait()

    iota = lax.iota(_i32, L)

    @plsc.parallel_loop(0, RPW, unroll=4)
    def _row(r):
        ctxb = plsc.load_gather(ctxv, [jnp.full((L,), r, _i32)])
        for j in range(D // L):
            sl = pl.ds(j * L, L)
            col = iota + (j * L)
            x = slab[r, sl]
            s = plsc.load_gather(sv, [ctxb, col])
            t = plsc.load_gather(tv, [ctxb, col])
            slab[r, sl] = x * s + t

    pltpu.sync_copy(slab, out_hbm.at[pl.ds(base, RPW), :])


@functools.cache
def _build():
    mesh = plsc.VectorSubcoreMesh(
        core_axis_name="c", subcore_axis_name="s", num_cores=NC, num_subcores=NS
    )
    params = pltpu.CompilerParams(needs_layout_passes=False)
    stats = pl.kernel(
        _stats,
        compiler_params=params,
        out_type=(
            jax.ShapeDtypeStruct((C, NW, D), _f32),
            jax.ShapeDtypeStruct((C, NW, D), _f32),
            jax.ShapeDtypeStruct((C, NW, L), _f32),
        ),
        mesh=mesh,
        scratch_types=[
            pltpu.VMEM((RPW, D), _f32),
            pltpu.VMEM((RPW,), _i32),
            pltpu.VMEM((C, D), _f32),
            pltpu.VMEM((C, D), _f32),
            pltpu.VMEM((C, L), _f32),
        ],
    )
    apply_ = pl.kernel(
        _apply,
        compiler_params=params,
        out_type=jax.ShapeDtypeStruct((N, D), _f32),
        mesh=mesh,
        scratch_types=[
            pltpu.VMEM((RPW, D), _f32),
            pltpu.VMEM((RPW,), _i32),
            pltpu.VMEM((C, D), _f32),
            pltpu.VMEM((C, D), _f32),
            pltpu.VMEM((2, NW, D), _f32),
            pltpu.VMEM((2, NW, D), _f32),
            pltpu.VMEM((2, NW, L), _f32),
            pltpu.VMEM((2, D), _f32),
            pltpu.VMEM((2, D), _f32),
            pltpu.VMEM((NCTX,), _f32),
            pltpu.VMEM((D,), _f32),
            pltpu.VMEM((D,), _f32),
            pltpu.VMEM_SHARED((C, D), _f32),
            pltpu.VMEM_SHARED((C, D), _f32),
            pltpu.SemaphoreType.DMA,
            pltpu.SemaphoreType.DMA,
            pltpu.SemaphoreType.DMA,
        ],
    )
    return stats, apply_


@jax.jit
def kernel(samples, contexts, gamma, beta, priors):
    _stats_k, _apply_k = _build()
    ctx = contexts.astype(_i32)
    s, q, n = _stats_k(samples, ctx)
    return _apply_k(samples, ctx, gamma, beta, priors, s, q, n)


# speedup vs baseline: 4.9861x; 1.0437x over previous
import functools

import jax
import jax.numpy as jnp
from jax import lax
from jax.experimental import pallas as pl
from jax.experimental.pallas import tpu as pltpu
from jax.experimental.pallas import tpu_sc as plsc

N = 16384
D = 128
NCTX = 26
C = 32
NC = 2
NS = 16
NW = NC * NS
RPW = N // NW
L = 16
EPS = 1e-3

_f32 = jnp.float32
_i32 = jnp.int32


def _rsqrt(x):
    i = plsc.bitcast(x, _i32)
    i = jnp.int32(0x5F3759DF) - lax.shift_right_logical(i, 1)
    y = plsc.bitcast(i, _f32)
    for _ in range(3):
        y = y * (1.5 - 0.5 * x * y * y)
    return y


def _wid():
    return lax.axis_index("s") * NC + lax.axis_index("c")


def _stats(samples_hbm, ctx_hbm, osum, osq, ocnt, slab, ctxv, accs, accq, accc):
    base = _wid() * RPW
    pltpu.sync_copy(samples_hbm.at[pl.ds(base, RPW), :], slab)
    pltpu.sync_copy(ctx_hbm.at[pl.ds(base, RPW)], ctxv)

    zeros = jnp.zeros((L,), _f32)

    @plsc.parallel_loop(0, C, unroll=4)
    def _zero(c):
        for j in range(D // L):
            accs[c, pl.ds(j * L, L)] = zeros
            accq[c, pl.ds(j * L, L)] = zeros
        accc[c, :] = zeros

    iota = lax.iota(_i32, L)
    ones = jnp.ones((L,), _f32)

    @plsc.parallel_loop(0, RPW, unroll=4)
    def _row(r):
        ctxb = plsc.load_gather(ctxv, [jnp.full((L,), r, _i32)])
        for j in range(D // L):
            x = slab[r, pl.ds(j * L, L)]
            col = iota + (j * L)
            plsc.addupdate_scatter(accs, [ctxb, col], x)
            plsc.addupdate_scatter(accq, [ctxb, col], x * x)
        plsc.addupdate_scatter(accc, [ctxb, iota], ones)

    wid = _wid()
    pltpu.sync_copy(accs, osum.at[:, wid, :])
    pltpu.sync_copy(accq, osq.at[:, wid, :])
    pltpu.sync_copy(accc, ocnt.at[:, wid, :])


def _apply(samples_hbm, ctx_hbm, gamma_hbm, beta_hbm, priors_hbm,
           psum, psq, pcnt, out_hbm,
           slab, ctxv, sv, bs, bq, bc, grow, brow, pv, srow,
           shs, sem_slab, sem_ctx, sem_cmb):
    core = lax.axis_index("c")
    sid = lax.axis_index("s")
    base = (sid * NC + core) * RPW
    cp_slab = pltpu.async_copy(samples_hbm.at[pl.ds(base, RPW), :], slab, sem_slab)
    cp_ctx = pltpu.async_copy(ctx_hbm.at[pl.ds(base, RPW)], ctxv, sem_ctx)

    cps = []
    for k in range(2):
        c = sid * 2 + k
        cg = jnp.minimum(c, NCTX - 1)
        cps.append(pltpu.async_copy(psum.at[c], bs.at[k], sem_cmb))
        cps.append(pltpu.async_copy(psq.at[c], bq.at[k], sem_cmb))
        cps.append(pltpu.async_copy(pcnt.at[c], bc.at[k], sem_cmb))
        cps.append(pltpu.async_copy(gamma_hbm.at[cg], grow.at[k], sem_cmb))
        cps.append(pltpu.async_copy(beta_hbm.at[cg], brow.at[k], sem_cmb))
    cps.append(pltpu.async_copy(priors_hbm, pv, sem_cmb))
    for cp in cps:
        cp.wait()

    zeros = jnp.zeros((L,), _f32)

    for k in range(2):
        c = sid * 2 + k
        cg = jnp.minimum(c, NCTX - 1)
        invp = _rsqrt(plsc.load_gather(pv, [jnp.full((L,), cg, _i32)]))

        def red_cnt(t, acc):
            return acc + bc[k, t, :]

        cnt = lax.fori_loop(0, NW, red_cnt, zeros, unroll=8)
        cnt = jnp.maximum(cnt, 1.0)

        for j in range(D // L):
            sl = pl.ds(j * L, L)

            def red(t, acc):
                a, q = acc
                return (a + bs[k, t, sl], q + bq[k, t, sl])

            a, q = lax.fori_loop(0, NW, red, (zeros, zeros), unroll=8)
            mean = a / cnt
            var = q / cnt - mean * mean
            inv = _rsqrt(var + EPS)
            s = grow[k, sl] * inv * invp
            t = brow[k, sl] * invp - mean * s
            rnd = jnp.int32(0x8000)
            si = lax.shift_right_logical(plsc.bitcast(s, _i32) + rnd, 16)
            ti = (plsc.bitcast(t, _i32) + rnd) & jnp.int32(-65536)
            srow[sl] = si | ti

        pltpu.sync_copy(srow, shs.at[c])

    plsc.subcore_barrier()
    pltpu.sync_copy(shs, sv)

    cp_slab.wait()
    cp_ctx.wait()

    iota = lax.iota(_i32, L)

    @plsc.parallel_loop(0, RPW, unroll=4)
    def _row(r):
        ctxb = plsc.load_gather(ctxv, [jnp.full((L,), r, _i32)])
        for j in range(D // L):
            sl = pl.ds(j * L, L)
            col = iota + (j * L)
            x = slab[r, sl]
            pk = plsc.load_gather(sv, [ctxb, col])
            s = plsc.bitcast(lax.shift_left(pk, 16), _f32)
            t = plsc.bitcast(pk & jnp.int32(-65536), _f32)
            slab[r, sl] = x * s + t

    pltpu.sync_copy(slab, out_hbm.at[pl.ds(base, RPW), :])


@functools.cache
def _build():
    mesh = plsc.VectorSubcoreMesh(
        core_axis_name="c", subcore_axis_name="s", num_cores=NC, num_subcores=NS
    )
    params = pltpu.CompilerParams(needs_layout_passes=False)
    stats = pl.kernel(
        _stats,
        compiler_params=params,
        out_type=(
            jax.ShapeDtypeStruct((C, NW, D), _f32),
            jax.ShapeDtypeStruct((C, NW, D), _f32),
            jax.ShapeDtypeStruct((C, NW, L), _f32),
        ),
        mesh=mesh,
        scratch_types=[
            pltpu.VMEM((RPW, D), _f32),
            pltpu.VMEM((RPW,), _i32),
            pltpu.VMEM((C, D), _f32),
            pltpu.VMEM((C, D), _f32),
            pltpu.VMEM((C, L), _f32),
        ],
    )
    apply_ = pl.kernel(
        _apply,
        compiler_params=params,
        out_type=jax.ShapeDtypeStruct((N, D), _f32),
        mesh=mesh,
        scratch_types=[
            pltpu.VMEM((RPW, D), _f32),
            pltpu.VMEM((RPW,), _i32),
            pltpu.VMEM((C, D), _i32),
            pltpu.VMEM((2, NW, D), _f32),
            pltpu.VMEM((2, NW, D), _f32),
            pltpu.VMEM((2, NW, L), _f32),
            pltpu.VMEM((2, D), _f32),
            pltpu.VMEM((2, D), _f32),
            pltpu.VMEM((NCTX,), _f32),
            pltpu.VMEM((D,), _i32),
            pltpu.VMEM_SHARED((C, D), _i32),
            pltpu.SemaphoreType.DMA,
            pltpu.SemaphoreType.DMA,
            pltpu.SemaphoreType.DMA,
        ],
    )
    return stats, apply_


@jax.jit
def kernel(samples, contexts, gamma, beta, priors):
    _stats_k, _apply_k = _build()
    ctx = contexts.astype(_i32)
    s, q, n = _stats_k(samples, ctx)
    return _apply_k(samples, ctx, gamma, beta, priors, s, q, n)


# speedup vs baseline: 5.2011x; 1.0431x over previous
import functools

import jax
import jax.numpy as jnp
from jax import lax
from jax.experimental import pallas as pl
from jax.experimental.pallas import tpu as pltpu
from jax.experimental.pallas import tpu_sc as plsc

N = 16384
D = 128
NCTX = 26
C = 32
NC = 2
NS = 16
NW = NC * NS
RPW = N // NW
L = 16
EPS = 1e-3

_f32 = jnp.float32
_i32 = jnp.int32


def _rsqrt(x):
    i = plsc.bitcast(x, _i32)
    i = jnp.int32(0x5F3759DF) - lax.shift_right_logical(i, 1)
    y = plsc.bitcast(i, _f32)
    for _ in range(3):
        y = y * (1.5 - 0.5 * x * y * y)
    return y


def _wid():
    return lax.axis_index("s") * NC + lax.axis_index("c")


def _stats(samples_hbm, ctx_hbm, osum, osq, ocnt, slab, ctxv, accs, accq, accc,
           sem_a, sem_b, sem_ctx):
    H = RPW // 2
    base = _wid() * RPW
    cp_a = pltpu.async_copy(samples_hbm.at[pl.ds(base, H), :],
                            slab.at[pl.ds(0, H)], sem_a)
    cp_b = pltpu.async_copy(samples_hbm.at[pl.ds(base + H, H), :],
                            slab.at[pl.ds(H, H)], sem_b)
    cp_ctx = pltpu.async_copy(ctx_hbm.at[pl.ds(base, RPW)], ctxv, sem_ctx)

    zeros = jnp.zeros((L,), _f32)

    @plsc.parallel_loop(0, C, unroll=4)
    def _zero(c):
        for j in range(D // L):
            accs[c, pl.ds(j * L, L)] = zeros
            accq[c, pl.ds(j * L, L)] = zeros
        accc[c, :] = zeros

    iota = lax.iota(_i32, L)
    ones = jnp.ones((L,), _f32)

    cp_ctx.wait()

    def _half_rows(lo, hi):
        @plsc.parallel_loop(lo, hi, unroll=4)
        def _row(r):
            ctxb = plsc.load_gather(ctxv, [jnp.full((L,), r, _i32)])
            for j in range(D // L):
                x = slab[r, pl.ds(j * L, L)]
                col = iota + (j * L)
                plsc.addupdate_scatter(accs, [ctxb, col], x)
                plsc.addupdate_scatter(accq, [ctxb, col], x * x)
            plsc.addupdate_scatter(accc, [ctxb, iota], ones)

    cp_a.wait()
    _half_rows(0, H)
    cp_b.wait()
    _half_rows(H, RPW)

    wid = _wid()
    o1 = pltpu.async_copy(accs, osum.at[:, wid, :], sem_a)
    o2 = pltpu.async_copy(accq, osq.at[:, wid, :], sem_b)
    o3 = pltpu.async_copy(accc, ocnt.at[:, wid, :], sem_ctx)
    o1.wait()
    o2.wait()
    o3.wait()


def _apply(samples_hbm, ctx_hbm, gamma_hbm, beta_hbm, priors_hbm,
           psum, psq, pcnt, out_hbm,
           slab, ctxv, sv, bs, bq, bc, grow, brow, pv, srow,
           shs, sem_slab, sem_ctx, sem_cmb, sem_out):
    core = lax.axis_index("c")
    sid = lax.axis_index("s")
    H = RPW // 2
    base = (sid * NC + core) * RPW
    cp_a = pltpu.async_copy(samples_hbm.at[pl.ds(base, H), :],
                            slab.at[pl.ds(0, H)], sem_slab)
    cp_b = pltpu.async_copy(samples_hbm.at[pl.ds(base + H, H), :],
                            slab.at[pl.ds(H, H)], sem_out)
    cp_ctx = pltpu.async_copy(ctx_hbm.at[pl.ds(base, RPW)], ctxv, sem_ctx)

    cps = []
    for k in range(2):
        c = sid * 2 + k
        cg = jnp.minimum(c, NCTX - 1)
        cps.append(pltpu.async_copy(psum.at[c], bs.at[k], sem_cmb))
        cps.append(pltpu.async_copy(psq.at[c], bq.at[k], sem_cmb))
        cps.append(pltpu.async_copy(pcnt.at[c], bc.at[k], sem_cmb))
        cps.append(pltpu.async_copy(gamma_hbm.at[cg], grow.at[k], sem_cmb))
        cps.append(pltpu.async_copy(beta_hbm.at[cg], brow.at[k], sem_cmb))
    cps.append(pltpu.async_copy(priors_hbm, pv, sem_cmb))
    for cp in cps:
        cp.wait()

    zeros = jnp.zeros((L,), _f32)

    for k in range(2):
        c = sid * 2 + k
        cg = jnp.minimum(c, NCTX - 1)
        invp = _rsqrt(plsc.load_gather(pv, [jnp.full((L,), cg, _i32)]))

        def red_cnt(t, acc):
            return acc + bc[k, t, :]

        cnt = lax.fori_loop(0, NW, red_cnt, zeros, unroll=8)
        cnt = jnp.maximum(cnt, 1.0)

        for j in range(D // L):
            sl = pl.ds(j * L, L)

            def red(t, acc):
                a, q = acc
                return (a + bs[k, t, sl], q + bq[k, t, sl])

            a, q = lax.fori_loop(0, NW, red, (zeros, zeros), unroll=8)
            mean = a / cnt
            var = q / cnt - mean * mean
            inv = _rsqrt(var + EPS)
            s = grow[k, sl] * inv * invp
            t = brow[k, sl] * invp - mean * s
            rnd = jnp.int32(0x8000)
            si = lax.shift_right_logical(plsc.bitcast(s, _i32) + rnd, 16)
            ti = (plsc.bitcast(t, _i32) + rnd) & jnp.int32(-65536)
            srow[sl] = si | ti

        pltpu.sync_copy(srow, shs.at[c])

    plsc.subcore_barrier()
    pltpu.sync_copy(shs, sv)

    cp_ctx.wait()

    iota = lax.iota(_i32, L)

    def _half_rows(lo, hi):
        @plsc.parallel_loop(lo, hi, unroll=4)
        def _row(r):
            ctxb = plsc.load_gather(ctxv, [jnp.full((L,), r, _i32)])
            for j in range(D // L):
                sl = pl.ds(j * L, L)
                col = iota + (j * L)
                x = slab[r, sl]
                pk = plsc.load_gather(sv, [ctxb, col])
                s = plsc.bitcast(lax.shift_left(pk, 16), _f32)
                t = plsc.bitcast(pk & jnp.int32(-65536), _f32)
                slab[r, sl] = x * s + t

    cp_a.wait()
    _half_rows(0, H)
    oa = pltpu.async_copy(slab.at[pl.ds(0, H)],
                          out_hbm.at[pl.ds(base, H), :], sem_slab)
    cp_b.wait()
    _half_rows(H, RPW)
    ob = pltpu.async_copy(slab.at[pl.ds(H, H)],
                          out_hbm.at[pl.ds(base + H, H), :], sem_out)
    oa.wait()
    ob.wait()


@functools.cache
def _build():
    mesh = plsc.VectorSubcoreMesh(
        core_axis_name="c", subcore_axis_name="s", num_cores=NC, num_subcores=NS
    )
    params = pltpu.CompilerParams(needs_layout_passes=False)
    stats = pl.kernel(
        _stats,
        compiler_params=params,
        out_type=(
            jax.ShapeDtypeStruct((C, NW, D), _f32),
            jax.ShapeDtypeStruct((C, NW, D), _f32),
            jax.ShapeDtypeStruct((C, NW, L), _f32),
        ),
        mesh=mesh,
        scratch_types=[
            pltpu.VMEM((RPW, D), _f32),
            pltpu.VMEM((RPW,), _i32),
            pltpu.VMEM((C, D), _f32),
            pltpu.VMEM((C, D), _f32),
            pltpu.VMEM((C, L), _f32),
            pltpu.SemaphoreType.DMA,
            pltpu.SemaphoreType.DMA,
            pltpu.SemaphoreType.DMA,
        ],
    )
    apply_ = pl.kernel(
        _apply,
        compiler_params=params,
        out_type=jax.ShapeDtypeStruct((N, D), _f32),
        mesh=mesh,
        scratch_types=[
            pltpu.VMEM((RPW, D), _f32),
            pltpu.VMEM((RPW,), _i32),
            pltpu.VMEM((C, D), _i32),
            pltpu.VMEM((2, NW, D), _f32),
            pltpu.VMEM((2, NW, D), _f32),
            pltpu.VMEM((2, NW, L), _f32),
            pltpu.VMEM((2, D), _f32),
            pltpu.VMEM((2, D), _f32),
            pltpu.VMEM((NCTX,), _f32),
            pltpu.VMEM((D,), _i32),
            pltpu.VMEM_SHARED((C, D), _i32),
            pltpu.SemaphoreType.DMA,
            pltpu.SemaphoreType.DMA,
            pltpu.SemaphoreType.DMA,
            pltpu.SemaphoreType.DMA,
        ],
    )
    return stats, apply_


@jax.jit
def kernel(samples, contexts, gamma, beta, priors):
    _stats_k, _apply_k = _build()
    ctx = contexts.astype(_i32)
    s, q, n = _stats_k(samples, ctx)
    return _apply_k(samples, ctx, gamma, beta, priors, s, q, n)
